# baseline (speedup 1.0000x reference)
import functools

import jax
import jax.numpy as jnp
from jax import lax
from jax.experimental import pallas as pl
from jax.experimental.pallas import tpu as pltpu
from jax.experimental.pallas import tpu_sc as plsc

NPTS = 20000
KNN = 16
CHUNK_PTS = 32
CHUNK_EDGES = CHUNK_PTS * KNN
NSPLIT = -(-CHUNK_EDGES // 128)
NCHUNKS = NPTS // CHUNK_PTS
NWORKERS = 32
DEPTH = 2
VITERS = -(-(-(-NCHUNKS // NWORKERS)) // DEPTH) * DEPTH
CTR0 = CHUNK_EDGES
CTR_F = CHUNK_PTS + 16
META_W = CHUNK_EDGES + 3 * CTR_F

_OUT_D = 128


def _build_sc_call():
    mesh = plsc.VectorSubcoreMesh(core_axis_name="c", subcore_axis_name="s")

    @functools.partial(
        pl.kernel,
        mesh=mesh,
        out_type=jax.ShapeDtypeStruct((NPTS, _OUT_D), jnp.float32),
        compiler_params=pltpu.CompilerParams(use_tc_tiling_on_sc=False),
        scratch_types=[
            pltpu.VMEM((DEPTH, META_W), jnp.int32),
            pltpu.VMEM((DEPTH, CHUNK_EDGES, 64), jnp.float32),
            pltpu.VMEM((DEPTH, CHUNK_EDGES), jnp.float32),
            pltpu.VMEM((DEPTH, CHUNK_EDGES), jnp.float32),
            pltpu.VMEM((DEPTH, CHUNK_EDGES), jnp.float32),
            pltpu.VMEM((4, 64), jnp.float32),
            pltpu.VMEM((64,), jnp.float32),
            pltpu.VMEM((DEPTH, CHUNK_PTS, _OUT_D), jnp.float32),
        ] + [pltpu.SemaphoreType.DMA] * (3 * DEPTH),
    )
    def lfa_kernel(meta_hbm, feat_hbm, px_hbm, py_hbm, pz_hbm, wt_hbm, b_hbm,
                   out_hbm,
                   meta_v, featbuf, nbx, nby, nbz, wtbuf, bbuf, outbuf,
                   *sems):
        sem_m = sems[0:DEPTH]
        sem_g = sems[DEPTH:2 * DEPTH]
        sem_o = sems[2 * DEPTH:3 * DEPTH]
        wid = lax.axis_index("s") * 2 + lax.axis_index("c")
        pltpu.sync_copy(wt_hbm, wtbuf)
        pltpu.sync_copy(b_hbm, bbuf)
        wvec = [[wtbuf[ci, pl.ds(16 * v, 16)] for ci in range(4)]
                for v in range(4)]
        bvec = [bbuf[pl.ds(16 * v, 16)] for v in range(4)]
        b06 = [bv * jnp.float32(0.6) for bv in bvec]

        def meta_copy(b, c):
            return pltpu.make_async_copy(
                meta_hbm.at[pl.ds(c * META_W, META_W)],
                meta_v.at[b], sem_m[b])

        def gather_copies(b):
            cps = []
            for t in range(NSPLIT):
                idx_ref = meta_v.at[b, pl.ds(t * 128, 128)]
                sl = pl.ds(t * 128, 128)
                cps.extend([
                    pltpu.make_async_copy(feat_hbm.at[idx_ref],
                                          featbuf.at[b, sl], sem_g[b]),
                    pltpu.make_async_copy(px_hbm.at[idx_ref],
                                          nbx.at[b, sl], sem_g[b]),
                    pltpu.make_async_copy(py_hbm.at[idx_ref],
                                          nby.at[b, sl], sem_g[b]),
                    pltpu.make_async_copy(pz_hbm.at[idx_ref],
                                          nbz.at[b, sl], sem_g[b]),
                ])
            return cps

        def out_copy(b, c):
            return pltpu.make_async_copy(
                outbuf.at[b],
                out_hbm.at[pl.ds(c * CHUNK_PTS, CHUNK_PTS)], sem_o[b])

        def compute_chunk(b, c, j, refill):
            refill()

            @pl.when(j >= DEPTH)
            def _():
                out_copy(b, c).wait()

            @plsc.parallel_loop(0, CHUNK_PTS, 1, unroll=4)
            def point_body(p):
                cx = meta_v[b, pl.ds(CTR0 + p, 16)][0]
                cy = meta_v[b, pl.ds(CTR0 + CTR_F + p, 16)][0]
                cz = meta_v[b, pl.ds(CTR0 + 2 * CTR_F + p, 16)][0]
                nx = nbx[b, pl.ds(p * KNN, KNN)]
                ny = nby[b, pl.ds(p * KNN, KNN)]
                nz = nbz[b, pl.ds(p * KNN, KNN)]
                dx = lax.bitcast_convert_type(cx, jnp.float32) - nx
                dy = lax.bitcast_convert_type(cy, jnp.float32) - ny
                dz = lax.bitcast_convert_type(cz, jnp.float32) - nz
                s = dx * dx + dy * dy + dz * dz
                bits = lax.bitcast_convert_type(s, jnp.int32)
                seed = jnp.int32(0x5F3759DF) - (bits >> 1)
                r = lax.bitcast_convert_type(seed, jnp.float32)
                hs = s * jnp.float32(-0.5)
                r = r * (hs * r * r + jnp.float32(1.5))
                r = r * (hs * r * r + jnp.float32(1.5))
                r = r * (hs * r * r + jnp.float32(1.5))
                nr = s * r
                acc_s = [jnp.zeros((16,), jnp.float32) for _ in range(4)]
                acc_a = [jnp.zeros((16,), jnp.float32) for _ in range(4)]
                acc_f = [jnp.zeros((16,), jnp.float32) for _ in range(4)]
                base = p * KNN
                for k in range(KNN):
                    dxk = dx[k]
                    dyk = dy[k]
                    dzk = dz[k]
                    nrk = nr[k]
                    for v in range(4):
                        z = (dxk * wvec[v][0] + dyk * wvec[v][1]
                             + dzk * wvec[v][2] + nrk * wvec[v][3]
                             + bvec[v])
                        acc_s[v] = acc_s[v] + z
                        acc_a[v] = acc_a[v] + jnp.abs(z)
                        acc_f[v] = acc_f[v] + featbuf[b, base + k,
                                                      pl.ds(16 * v, 16)]
                for v in range(4):
                    outbuf[b, p, pl.ds(16 * v, 16)] = (
                        acc_s[v] * jnp.float32(0.6 / KNN)
                        + acc_a[v] * jnp.float32(0.4 / KNN))
                    outbuf[b, p, pl.ds(64 + 16 * v, 16)] = (
                        acc_f[v] * jnp.float32(1.0 / KNN))

            out_copy(b, c).start()

        for d in range(DEPTH):
            meta_copy(d, wid + d * NWORKERS).start()
        for d in range(DEPTH - 1):
            meta_copy(d, wid + d * NWORKERS).wait()
            for cp in gather_copies(d):
                cp.start()

        def outer_body(o, carry):
            for bb in range(DEPTH):
                j = DEPTH * o + bb
                c = wid + NWORKERS * j
                c_new = c + (DEPTH - 1) * NWORKERS
                c_ref = c + DEPTH * NWORKERS
                b_new = (bb + DEPTH - 1) % DEPTH

                @pl.when(c_new < NCHUNKS)
                def _(b_new=b_new, c_new=c_new):
                    meta_copy(b_new, c_new).wait()
                    for cp in gather_copies(b_new):
                        cp.start()

                @pl.when(c < NCHUNKS)
                def _(bb=bb, c=c, c_ref=c_ref, j=j):
                    for cp in gather_copies(bb):
                        cp.wait()

                    def refill(bb=bb, c_ref=c_ref):
                        pass

                    compute_chunk(bb, c, j, refill)

                    @pl.when(c_ref < NCHUNKS)
                    def _():
                        meta_copy(bb, c_ref).start()
            return carry

        lax.fori_loop(0, VITERS // DEPTH, outer_body, 0)
        for d in range(DEPTH):
            out_copy(d, wid).wait()

    return lfa_kernel


_SC_CALL = _build_sc_call()


def kernel(points, features, knn_indices, W, b):
    B, N, D = points.shape
    pts = points.reshape(B * N, D).astype(jnp.float32)
    feat_flat = features.reshape(B * N, features.shape[-1]).astype(jnp.float32)
    px = pts[:, 0]
    py = pts[:, 1]
    pz = pts[:, 2]
    offs = (jnp.arange(B, dtype=jnp.int32) * N)[:, None, None]
    idx_flat = (knn_indices.astype(jnp.int32) + offs).reshape(NCHUNKS,
                                                              CHUNK_EDGES)
    zpad = jnp.zeros((NCHUNKS, CTR_F - CHUNK_PTS), jnp.int32)

    def cfield(v):
        return lax.bitcast_convert_type(v.reshape(NCHUNKS, CHUNK_PTS),
                                        jnp.int32)

    meta = jnp.concatenate(
        [idx_flat, cfield(px), zpad, cfield(py), zpad, cfield(pz), zpad],
        axis=1).reshape(-1)
    wt = W.astype(jnp.float32).T
    out = _SC_CALL(meta, feat_flat, px, py, pz, wt, b.astype(jnp.float32))
    return out.reshape(B, N, _OUT_D)


# speedup vs baseline: 1.4878x; 1.4878x over previous
import functools

import jax
import jax.numpy as jnp
from jax import lax
from jax.experimental import pallas as pl
from jax.experimental.pallas import tpu as pltpu
from jax.experimental.pallas import tpu_sc as plsc

NPTS = 20000
KNN = 16
CHUNK_PTS = 32
CHUNK_EDGES = CHUNK_PTS * KNN
NSPLIT = -(-CHUNK_EDGES // 128)
NCHUNKS = NPTS // CHUNK_PTS
NWORKERS = 32
DEPTH = 2
VITERS = -(-(-(-NCHUNKS // NWORKERS)) // DEPTH) * DEPTH
CTR0 = CHUNK_EDGES
CTR_F = CHUNK_PTS + 16
META_W = CHUNK_EDGES + 3 * CTR_F

_OUT_D = 128


def _build_sc_call():
    mesh = plsc.VectorSubcoreMesh(core_axis_name="c", subcore_axis_name="s")

    @functools.partial(
        pl.kernel,
        mesh=mesh,
        out_type=jax.ShapeDtypeStruct((NPTS, _OUT_D), jnp.float32),
        compiler_params=pltpu.CompilerParams(use_tc_tiling_on_sc=False),
        scratch_types=[
            pltpu.VMEM((DEPTH, META_W), jnp.int32),
            pltpu.VMEM((DEPTH, CHUNK_EDGES, 64), jnp.float32),
            pltpu.VMEM((DEPTH, CHUNK_EDGES), jnp.float32),
            pltpu.VMEM((DEPTH, CHUNK_EDGES), jnp.float32),
            pltpu.VMEM((DEPTH, CHUNK_EDGES), jnp.float32),
            pltpu.VMEM((4, 64), jnp.float32),
            pltpu.VMEM((64,), jnp.float32),
            pltpu.VMEM((DEPTH, CHUNK_PTS, _OUT_D), jnp.float32),
        ] + [pltpu.SemaphoreType.DMA] * (3 * DEPTH),
    )
    def lfa_kernel(meta_hbm, feat_hbm, px_hbm, py_hbm, pz_hbm, wt_hbm, b_hbm,
                   out_hbm,
                   meta_v, featbuf, nbx, nby, nbz, wtbuf, bbuf, outbuf,
                   *sems):
        sem_m = sems[0:DEPTH]
        sem_g = sems[DEPTH:2 * DEPTH]
        sem_o = sems[2 * DEPTH:3 * DEPTH]
        wid = lax.axis_index("s") * 2 + lax.axis_index("c")
        pltpu.sync_copy(wt_hbm, wtbuf)
        pltpu.sync_copy(b_hbm, bbuf)
        wvec = [[wtbuf[ci, pl.ds(16 * v, 16)] for ci in range(4)]
                for v in range(4)]
        bvec = [bbuf[pl.ds(16 * v, 16)] for v in range(4)]

        def meta_copy(b, c):
            return pltpu.make_async_copy(
                meta_hbm.at[pl.ds(c * META_W, META_W)],
                meta_v.at[b], sem_m[b])

        def gather_copies(b):
            cps = []
            for t in range(NSPLIT):
                idx_ref = meta_v.at[b, pl.ds(t * 128, 128)]
                sl = pl.ds(t * 128, 128)
                cps.extend([
                    pltpu.make_async_copy(feat_hbm.at[idx_ref],
                                          featbuf.at[b, sl], sem_g[b]),
                    pltpu.make_async_copy(px_hbm.at[idx_ref],
                                          nbx.at[b, sl], sem_g[b]),
                    pltpu.make_async_copy(py_hbm.at[idx_ref],
                                          nby.at[b, sl], sem_g[b]),
                    pltpu.make_async_copy(pz_hbm.at[idx_ref],
                                          nbz.at[b, sl], sem_g[b]),
                ])
            return cps

        def out_copy(b, c):
            return pltpu.make_async_copy(
                outbuf.at[b],
                out_hbm.at[pl.ds(c * CHUNK_PTS, CHUNK_PTS)], sem_o[b])

        def compute_chunk(b, c, j):
            @pl.when(j >= DEPTH)
            def _():
                out_copy(b, c).wait()

            @plsc.parallel_loop(0, CHUNK_PTS, 1, unroll=2)
            def point_body(p):
                cx = meta_v[b, pl.ds(CTR0 + p, 16)][0]
                cy = meta_v[b, pl.ds(CTR0 + CTR_F + p, 16)][0]
                cz = meta_v[b, pl.ds(CTR0 + 2 * CTR_F + p, 16)][0]
                nx = nbx[b, pl.ds(p * KNN, KNN)]
                ny = nby[b, pl.ds(p * KNN, KNN)]
                nz = nbz[b, pl.ds(p * KNN, KNN)]
                dx = lax.bitcast_convert_type(cx, jnp.float32) - nx
                dy = lax.bitcast_convert_type(cy, jnp.float32) - ny
                dz = lax.bitcast_convert_type(cz, jnp.float32) - nz
                s = dx * dx + dy * dy + dz * dz
                bits = lax.bitcast_convert_type(s, jnp.int32)
                seed = jnp.int32(0x5F3759DF) - (bits >> 1)
                r = lax.bitcast_convert_type(seed, jnp.float32)
                hs = s * jnp.float32(-0.5)
                r = r * (hs * r * r + jnp.float32(1.5))
                r = r * (hs * r * r + jnp.float32(1.5))
                r = r * (hs * r * r + jnp.float32(1.5))
                nr = s * r
                acc_s = [jnp.zeros((16,), jnp.float32) for _ in range(4)]
                acc_a = [jnp.zeros((16,), jnp.float32) for _ in range(4)]
                acc_f = [jnp.zeros((16,), jnp.float32) for _ in range(4)]
                base = p * KNN
                for k in range(KNN):
                    dxk = dx[k]
                    dyk = dy[k]
                    dzk = dz[k]
                    nrk = nr[k]
                    for v in range(4):
                        z = (dxk * wvec[v][0] + dyk * wvec[v][1]
                             + dzk * wvec[v][2] + nrk * wvec[v][3]
                             + bvec[v])
                        acc_s[v] = acc_s[v] + z
                        acc_a[v] = acc_a[v] + jnp.abs(z)
                        acc_f[v] = acc_f[v] + featbuf[b, base + k,
                                                      pl.ds(16 * v, 16)]
                for v in range(4):
                    outbuf[b, p, pl.ds(16 * v, 16)] = (
                        acc_s[v] * jnp.float32(0.6 / KNN)
                        + acc_a[v] * jnp.float32(0.4 / KNN))
                    outbuf[b, p, pl.ds(64 + 16 * v, 16)] = (
                        acc_f[v] * jnp.float32(1.0 / KNN))

            out_copy(b, c).start()

        for d in range(DEPTH):
            meta_copy(d, wid + d * NWORKERS).start()
        for d in range(DEPTH - 1):
            meta_copy(d, wid + d * NWORKERS).wait()
            for cp in gather_copies(d):
                cp.start()

        def outer_body(o, carry):
            for bb in range(DEPTH):
                j = DEPTH * o + bb
                c = wid + NWORKERS * j
                c_new = c + (DEPTH - 1) * NWORKERS
                c_ref = c + DEPTH * NWORKERS
                b_new = (bb + DEPTH - 1) % DEPTH

                @pl.when(c_new < NCHUNKS)
                def _(b_new=b_new, c_new=c_new):
                    meta_copy(b_new, c_new).wait()
                    for cp in gather_copies(b_new):
                        cp.start()

                @pl.when(c < NCHUNKS)
                def _(bb=bb, c=c, c_ref=c_ref, j=j):
                    for cp in gather_copies(bb):
                        cp.wait()

                    compute_chunk(bb, c, j)

                    @pl.when(c_ref < NCHUNKS)
                    def _():
                        meta_copy(bb, c_ref).start()
            return carry

        lax.fori_loop(0, VITERS // DEPTH, outer_body, 0)
        for d in range(DEPTH):
            out_copy(d, wid).wait()

    return lfa_kernel


_SC_CALL = _build_sc_call()


def kernel(points, features, knn_indices, W, b):
    B, N, D = points.shape
    pts = points.reshape(B * N, D).astype(jnp.float32)
    feat_flat = features.reshape(B * N, features.shape[-1]).astype(jnp.float32)
    px = pts[:, 0]
    py = pts[:, 1]
    pz = pts[:, 2]
    offs = (jnp.arange(B, dtype=jnp.int32) * N)[:, None, None]
    idx_flat = (knn_indices.astype(jnp.int32) + offs).reshape(NCHUNKS,
                                                              CHUNK_EDGES)
    zpad = jnp.zeros((NCHUNKS, CTR_F - CHUNK_PTS), jnp.int32)

    def cfield(v):
        return lax.bitcast_convert_type(v.reshape(NCHUNKS, CHUNK_PTS),
                                        jnp.int32)

    meta = jnp.concatenate(
        [idx_flat, cfield(px), zpad, cfield(py), zpad, cfield(pz), zpad],
        axis=1).reshape(-1)
    wt = W.astype(jnp.float32).T
    out = _SC_CALL(meta, feat_flat, px, py, pz, wt, b.astype(jnp.float32))
    return out.reshape(B, N, _OUT_D)
